# baseline (speedup 1.0000x reference)
import functools

import jax
import jax.numpy as jnp
from jax import lax
from jax.experimental import pallas as pl
from jax.experimental.pallas import tpu as pltpu
from jax.experimental.pallas import tpu_sc as plsc

NUM_POS = 8192
D_MODEL = 768
BATCH = 4
SEQ = 8192

NUM_CORES = 2
NUM_SUBCORES = 16
NUM_WORKERS = NUM_CORES * NUM_SUBCORES
SEQ_PER_W = SEQ // NUM_WORKERS
CHUNK = 64
NCHUNK = SEQ_PER_W // CHUNK
LANES = 16
NVEC = D_MODEL // LANES

_mesh = plsc.VectorSubcoreMesh(
    core_axis_name="c", subcore_axis_name="s")


@functools.partial(
    pl.kernel,
    mesh=_mesh,
    out_type=jax.ShapeDtypeStruct((BATCH * SEQ, D_MODEL), jnp.float32),
    scratch_types=[
        pltpu.VMEM((CHUNK,), jnp.int32),
        pltpu.VMEM((CHUNK, D_MODEL), jnp.float32),
        pltpu.VMEM((CHUNK, D_MODEL), jnp.float32),
        pltpu.SemaphoreType.DMA,
    ],
)
def _pos_enc_sc(x_hbm, pos_hbm, p_hbm, out_hbm, idx_v, p_v, x_v, sem):
    wid = lax.axis_index("s") * NUM_CORES + lax.axis_index("c")
    base = wid * SEQ_PER_W

    def chunk_body(c, carry):
        row0 = base + c * CHUNK
        pltpu.sync_copy(pos_hbm.at[pl.ds(row0, CHUNK)], idx_v)
        pltpu.async_copy(p_hbm.at[idx_v], p_v, sem).wait()

        def batch_body(b, carry2):
            xrow0 = b * SEQ + row0
            pltpu.sync_copy(x_hbm.at[pl.ds(xrow0, CHUNK)], x_v)

            def row_body(r, carry3):
                for j in range(NVEC):
                    sl = pl.ds(j * LANES, LANES)
                    x_v[r, sl] = x_v[r, sl] + p_v[r, sl]
                return carry3

            lax.fori_loop(0, CHUNK, row_body, 0)
            pltpu.sync_copy(x_v, out_hbm.at[pl.ds(xrow0, CHUNK)])
            return carry2

        lax.fori_loop(0, BATCH, batch_body, 0)
        return carry

    lax.fori_loop(0, NCHUNK, chunk_body, 0)


def kernel(X, pos, P):
    out = _pos_enc_sc(X.reshape(BATCH * SEQ, D_MODEL), pos, P)
    return out.reshape(BATCH, SEQ, D_MODEL)
